# baseline (speedup 1.0000x reference)
import jax
import jax.numpy as jnp
from jax import lax
from jax.experimental import pallas as pl

_L = 50
_LP = 56
_D = 128
_V = 32
_CPAD = 16


def _body(yp_ref, ch_ref, mz_ref, pe_ref, emb_ref, chw_ref,
          g1_ref, b1_ref, g2_ref, b2_ref, mzd_ref, out_ref):
    bb = out_ref.shape[0]
    rows = bb * _LP

    t = pe_ref[...] + emb_ref[...]
    mu = jnp.mean(t, axis=-1, keepdims=True)
    var = jnp.mean((t - mu) * (t - mu), axis=-1, keepdims=True)
    fused = (t - mu) / jnp.sqrt(var + 1e-5) * g1_ref[...] + b1_ref[...]

    ch = ch_ref[...].reshape(bb)[:, None]
    oc = (ch == lax.broadcasted_iota(jnp.int32, (bb, _CPAD), 1))
    cemb = jnp.dot(oc.astype(jnp.float32), chw_ref[...],
                   preferred_element_type=jnp.float32)
    inp = jnp.floor(mz_ref[...].reshape(bb)[:, None] / 0.001)
    arg = inp * mzd_ref[...]
    par = lax.broadcasted_iota(jnp.int32, (bb, _D), 1) % 2
    mzpe = jnp.where(par == 0, jnp.sin(arg), jnp.cos(arg))
    bits = lax.bitcast_convert_type(mzpe, jnp.int32)
    bits = bits + 0x0FFF + ((bits >> 13) & 1)
    mzpe = lax.bitcast_convert_type(bits & jnp.int32(-8192), jnp.float32)
    extra = cemb + mzpe

    mf = jnp.mean(fused, axis=-1, keepdims=True)
    fc = fused - mf
    vf = jnp.mean(fc * fc, axis=-1, keepdims=True)
    me = jnp.mean(extra, axis=-1, keepdims=True)
    ec = extra - me
    ve = jnp.mean(ec * ec, axis=-1, keepdims=True)
    cross = lax.dot_general(ec, fc, (((1,), (1,)), ((), ())),
                            preferred_element_type=jnp.float32)
    rr = lax.rsqrt(ve + vf.reshape(1, _V) + (2.0 / _D) * cross + 1e-5)

    g2 = g2_ref[...]
    b2 = b2_ref[...]
    fc_g = fc * g2
    ec_g = ec * g2
    ec_g56 = jnp.broadcast_to(ec_g[:, None, :], (bb, _LP, _D)).reshape(rows, _D)
    rr56 = jnp.broadcast_to(rr[:, None, :], (bb, _LP, _V)).reshape(rows, _V)

    zpad = jnp.zeros((_LP - _L, 1), jnp.int32)
    tok = jnp.concatenate(
        [jnp.concatenate([yp_ref[b, :][:, None], zpad], axis=0)
         for b in range(bb)], axis=0)
    iota_v = lax.broadcasted_iota(jnp.int32, (rows, _V), 1)
    ots = jnp.where(tok == iota_v, rr56, 0.0)
    pep = jnp.dot(ots, fc_g, preferred_element_type=jnp.float32)
    rrep = jnp.dot(ots, jnp.ones((_V, _D), jnp.float32),
                   preferred_element_type=jnp.float32)
    res = (pep + ec_g56 * rrep + b2).reshape(bb, _LP, _D)
    out_ref[...] = res[:, :_L, :]


def kernel(y, charge, mz, emb_w, charge_w, ln1_g, ln1_b, ln2_g, ln2_b,
           pe_table, mz_div):
    B, L = y.shape
    D = emb_w.shape[1]
    BB = 128
    grid = B // BB

    pe32 = pe_table[:_V]
    chw = jnp.zeros((_CPAD, D), jnp.float32).at[:charge_w.shape[0]].set(charge_w)
    mzd = jnp.repeat(mz_div, 2).reshape(1, D)
    return pl.pallas_call(
        _body,
        grid=(grid,),
        in_specs=[
            pl.BlockSpec((BB, L), lambda i: (i, 0)),
            pl.BlockSpec((BB,), lambda i: (i,)),
            pl.BlockSpec((BB,), lambda i: (i,)),
            pl.BlockSpec((_V, D), lambda i: (0, 0)),
            pl.BlockSpec((_V, D), lambda i: (0, 0)),
            pl.BlockSpec((_CPAD, D), lambda i: (0, 0)),
            pl.BlockSpec((1, D), lambda i: (0, 0)),
            pl.BlockSpec((1, D), lambda i: (0, 0)),
            pl.BlockSpec((1, D), lambda i: (0, 0)),
            pl.BlockSpec((1, D), lambda i: (0, 0)),
            pl.BlockSpec((1, D), lambda i: (0, 0)),
        ],
        out_specs=pl.BlockSpec((BB, L, D), lambda i: (i, 0, 0)),
        out_shape=jax.ShapeDtypeStruct((B, L, D), jnp.float32),
    )(y.astype(jnp.int32), charge.astype(jnp.int32), mz, pe32, emb_w, chw,
      ln1_g.reshape(1, D), ln1_b.reshape(1, D),
      ln2_g.reshape(1, D), ln2_b.reshape(1, D), mzd)


# speedup vs baseline: 1.7141x; 1.7141x over previous
import jax
import jax.numpy as jnp
from jax import lax
from jax.experimental import pallas as pl

_L = 50
_D = 128
_V = 32
_CPAD = 16


def _body(y_ref, ch_ref, mz_ref, pe_ref, emb_ref, chw_ref,
          g1_ref, b1_ref, g2_ref, b2_ref, mzd_ref, out_ref):
    bb = out_ref.shape[1]
    rows = _L * bb

    t = pe_ref[...] + emb_ref[...]
    mu = jnp.mean(t, axis=-1, keepdims=True)
    var = jnp.mean((t - mu) * (t - mu), axis=-1, keepdims=True)
    fused = (t - mu) / jnp.sqrt(var + 1e-5) * g1_ref[...] + b1_ref[...]

    ch = ch_ref[...].reshape(bb)[:, None]
    oc = (ch == lax.broadcasted_iota(jnp.int32, (bb, _CPAD), 1))
    cemb = jnp.dot(oc.astype(jnp.float32), chw_ref[...],
                   preferred_element_type=jnp.float32)
    inp = jnp.floor(mz_ref[...].reshape(bb)[:, None] / 0.001)
    arg = inp * mzd_ref[...]
    par = lax.broadcasted_iota(jnp.int32, (bb, _D), 1) % 2
    mzpe = jnp.where(par == 0, jnp.sin(arg), jnp.cos(arg))
    bits = lax.bitcast_convert_type(mzpe, jnp.int32)
    bits = bits + 0x0FFF + ((bits >> 13) & 1)
    mzpe = lax.bitcast_convert_type(bits & jnp.int32(-8192), jnp.float32)
    extra = cemb + mzpe

    mf = jnp.mean(fused, axis=-1, keepdims=True)
    fc = fused - mf
    vf = jnp.mean(fc * fc, axis=-1, keepdims=True)
    me = jnp.mean(extra, axis=-1, keepdims=True)
    ec = extra - me
    ve = jnp.mean(ec * ec, axis=-1, keepdims=True)
    cross = lax.dot_general(ec, fc, (((1,), (1,)), ((), ())),
                            preferred_element_type=jnp.float32)
    rr = lax.rsqrt(ve + vf.reshape(1, _V) + (2.0 / _D) * cross + 1e-5)

    g2 = g2_ref[...]
    b2 = b2_ref[...]
    fc_g = fc * g2
    ec_g = ec * g2
    rhs = jnp.concatenate([fc_g, jnp.ones((_V, _D), jnp.float32)], axis=1)

    tok3 = jnp.stack([y_ref[:, l:l + 1] for l in range(_L)], axis=0)
    iota3 = lax.broadcasted_iota(jnp.int32, (_L, bb, _V), 2)
    ots = jnp.where(tok3 == iota3, rr[None, :, :], 0.0).reshape(rows, _V)
    big = jnp.dot(ots, rhs, preferred_element_type=jnp.float32)
    pep = big[:, :_D].reshape(_L, bb, _D)
    rrep = big[:, _D:].reshape(_L, bb, _D)
    out_ref[...] = pep + ec_g[None, :, :] * rrep + b2[None, :, :]


def kernel(y, charge, mz, emb_w, charge_w, ln1_g, ln1_b, ln2_g, ln2_b,
           pe_table, mz_div):
    B, L = y.shape
    D = emb_w.shape[1]
    BB = 128
    grid = B // BB

    pe32 = pe_table[:_V]
    chw = jnp.zeros((_CPAD, D), jnp.float32).at[:charge_w.shape[0]].set(charge_w)
    mzd = jnp.repeat(mz_div, 2).reshape(1, D)

    out = pl.pallas_call(
        _body,
        grid=(grid,),
        in_specs=[
            pl.BlockSpec((BB, L), lambda i: (i, 0)),
            pl.BlockSpec((BB,), lambda i: (i,)),
            pl.BlockSpec((BB,), lambda i: (i,)),
            pl.BlockSpec((_V, D), lambda i: (0, 0)),
            pl.BlockSpec((_V, D), lambda i: (0, 0)),
            pl.BlockSpec((_CPAD, D), lambda i: (0, 0)),
            pl.BlockSpec((1, D), lambda i: (0, 0)),
            pl.BlockSpec((1, D), lambda i: (0, 0)),
            pl.BlockSpec((1, D), lambda i: (0, 0)),
            pl.BlockSpec((1, D), lambda i: (0, 0)),
            pl.BlockSpec((1, D), lambda i: (0, 0)),
        ],
        out_specs=pl.BlockSpec((L, BB, D), lambda i: (0, i, 0)),
        out_shape=jax.ShapeDtypeStruct((L, B, D), jnp.float32),
    )(y.astype(jnp.int32), charge.astype(jnp.int32), mz, pe32, emb_w, chw,
      ln1_g.reshape(1, D), ln1_b.reshape(1, D),
      ln2_g.reshape(1, D), ln2_b.reshape(1, D), mzd)
    return jnp.transpose(out, (1, 0, 2))


# speedup vs baseline: 1.7472x; 1.0194x over previous
import jax
import jax.numpy as jnp
from jax import lax
from jax.experimental import pallas as pl

_L = 50
_D = 128
_V = 32
_CPAD = 16


def _body(y_ref, ch_ref, mz_ref, pe_ref, emb_ref, chw_ref,
          g1_ref, b1_ref, g2_ref, b2_ref, mzd_ref, out_ref):
    bb = out_ref.shape[1]
    rows = _L * bb

    t = pe_ref[...] + emb_ref[...]
    mu = jnp.mean(t, axis=-1, keepdims=True)
    var = jnp.mean((t - mu) * (t - mu), axis=-1, keepdims=True)
    fused = (t - mu) / jnp.sqrt(var + 1e-5) * g1_ref[...] + b1_ref[...]

    ch = ch_ref[...].reshape(bb)[:, None]
    oc = (ch == lax.broadcasted_iota(jnp.int32, (bb, _CPAD), 1))
    cemb = jnp.dot(oc.astype(jnp.float32), chw_ref[...],
                   preferred_element_type=jnp.float32)
    inp = jnp.floor(mz_ref[...].reshape(bb)[:, None] / 0.001)
    arg = inp * mzd_ref[...]
    par = lax.broadcasted_iota(jnp.int32, (bb, _D), 1) % 2
    mzpe = jnp.where(par == 0, jnp.sin(arg), jnp.cos(arg))
    bits = lax.bitcast_convert_type(mzpe, jnp.int32)
    bits = bits + 0x0FFF + ((bits >> 13) & 1)
    mzpe = lax.bitcast_convert_type(bits & jnp.int32(-8192), jnp.float32)
    extra = cemb + mzpe

    mf = jnp.mean(fused, axis=-1, keepdims=True)
    fc = fused - mf
    vf = jnp.mean(fc * fc, axis=-1, keepdims=True)
    me = jnp.mean(extra, axis=-1, keepdims=True)
    ec = extra - me
    ve = jnp.mean(ec * ec, axis=-1, keepdims=True)
    cross = lax.dot_general(ec, fc, (((1,), (1,)), ((), ())),
                            preferred_element_type=jnp.float32)
    rr = lax.rsqrt(ve + vf.reshape(1, _V) + (2.0 / _D) * cross + 1e-5)

    g2 = g2_ref[...]
    b2 = b2_ref[...]
    fc_g = fc * g2
    ec_g = ec * g2
    rhs = jnp.concatenate([fc_g, jnp.ones((_V, _D), jnp.float32)], axis=1)

    tok3 = jnp.stack([y_ref[:, l:l + 1] for l in range(_L)], axis=0)
    iota3 = lax.broadcasted_iota(jnp.int32, (_L, bb, _V), 2)
    ots = jnp.where(tok3 == iota3, rr[None, :, :], 0.0).reshape(rows, _V)
    big = jnp.dot(ots, rhs, preferred_element_type=jnp.float32)
    pep = big[:, :_D].reshape(_L, bb, _D)
    rrep = big[:, _D:].reshape(_L, bb, _D)
    out_ref[...] = pep + ec_g[None, :, :] * rrep + b2[None, :, :]


def kernel(y, charge, mz, emb_w, charge_w, ln1_g, ln1_b, ln2_g, ln2_b,
           pe_table, mz_div):
    B, L = y.shape
    D = emb_w.shape[1]
    BB = 256
    grid = B // BB

    pe32 = pe_table[:_V]
    chw = jnp.zeros((_CPAD, D), jnp.float32).at[:charge_w.shape[0]].set(charge_w)
    mzd = jnp.repeat(mz_div, 2).reshape(1, D)

    out = pl.pallas_call(
        _body,
        grid=(grid,),
        in_specs=[
            pl.BlockSpec((BB, L), lambda i: (i, 0)),
            pl.BlockSpec((BB,), lambda i: (i,)),
            pl.BlockSpec((BB,), lambda i: (i,)),
            pl.BlockSpec((_V, D), lambda i: (0, 0)),
            pl.BlockSpec((_V, D), lambda i: (0, 0)),
            pl.BlockSpec((_CPAD, D), lambda i: (0, 0)),
            pl.BlockSpec((1, D), lambda i: (0, 0)),
            pl.BlockSpec((1, D), lambda i: (0, 0)),
            pl.BlockSpec((1, D), lambda i: (0, 0)),
            pl.BlockSpec((1, D), lambda i: (0, 0)),
            pl.BlockSpec((1, D), lambda i: (0, 0)),
        ],
        out_specs=pl.BlockSpec((L, BB, D), lambda i: (0, i, 0)),
        out_shape=jax.ShapeDtypeStruct((L, B, D), jnp.float32),
    )(y.astype(jnp.int32), charge.astype(jnp.int32), mz, pe32, emb_w, chw,
      ln1_g.reshape(1, D), ln1_b.reshape(1, D),
      ln2_g.reshape(1, D), ln2_b.reshape(1, D), mzd)
    return jnp.transpose(out, (1, 0, 2))


# speedup vs baseline: 2.0075x; 1.1489x over previous
import jax
import jax.numpy as jnp
from jax import lax
from jax.experimental import pallas as pl

_L = 50
_D = 128
_V = 32
_CPAD = 16


def _body(y_ref, ch_ref, mz_ref, pe_ref, emb_ref, chw_ref,
          g1_ref, b1_ref, g2_ref, b2_ref, mzd_ref, out_ref):
    bb = out_ref.shape[1]
    rows = _L * bb

    t = pe_ref[...] + emb_ref[...]
    mu = jnp.mean(t, axis=-1, keepdims=True)
    var = jnp.mean((t - mu) * (t - mu), axis=-1, keepdims=True)
    fused = (t - mu) / jnp.sqrt(var + 1e-5) * g1_ref[...] + b1_ref[...]

    ch = ch_ref[...].reshape(bb)[:, None]
    oc = (ch == lax.broadcasted_iota(jnp.int32, (bb, _CPAD), 1))
    cemb = jnp.dot(oc.astype(jnp.float32), chw_ref[...],
                   preferred_element_type=jnp.float32)
    inp = jnp.floor(mz_ref[...].reshape(bb)[:, None] / 0.001)
    arg = inp * mzd_ref[...]
    par = lax.broadcasted_iota(jnp.int32, (bb, _D), 1) % 2
    mzpe = jnp.where(par == 0, jnp.sin(arg), jnp.cos(arg))
    bits = lax.bitcast_convert_type(mzpe, jnp.int32)
    bits = bits + 0x0FFF + ((bits >> 13) & 1)
    mzpe = lax.bitcast_convert_type(bits & jnp.int32(-8192), jnp.float32)
    extra = cemb + mzpe

    mf = jnp.mean(fused, axis=-1, keepdims=True)
    fc = fused - mf
    vf = jnp.mean(fc * fc, axis=-1, keepdims=True)
    me = jnp.mean(extra, axis=-1, keepdims=True)
    ec = extra - me
    ve = jnp.mean(ec * ec, axis=-1, keepdims=True)
    cross = lax.dot_general(ec, fc, (((1,), (1,)), ((), ())),
                            preferred_element_type=jnp.float32)
    rr = lax.rsqrt(ve + vf.reshape(1, _V) + (2.0 / _D) * cross + 1e-5)

    g2 = g2_ref[...]
    b2 = b2_ref[...]
    fc_g = fc * g2
    ec_g = ec * g2
    tok3 = jnp.stack([y_ref[:, l:l + 1] for l in range(_L)], axis=0)
    iota3 = lax.broadcasted_iota(jnp.int32, (_L, bb, _V), 2)
    ots = jnp.where(tok3 == iota3, rr[None, :, :], 0.0).reshape(rows, _V)
    pep = jnp.dot(ots, fc_g,
                  preferred_element_type=jnp.float32).reshape(_L, bb, _D)
    rrep = jnp.dot(ots, jnp.ones((_V, _D), jnp.float32),
                   preferred_element_type=jnp.float32).reshape(_L, bb, _D)
    out_ref[...] = pep + ec_g[None, :, :] * rrep + b2[None, :, :]


def kernel(y, charge, mz, emb_w, charge_w, ln1_g, ln1_b, ln2_g, ln2_b,
           pe_table, mz_div):
    B, L = y.shape
    D = emb_w.shape[1]
    BB = 256
    grid = B // BB

    pe32 = pe_table[:_V]
    chw = jnp.zeros((_CPAD, D), jnp.float32).at[:charge_w.shape[0]].set(charge_w)
    mzd = jnp.repeat(mz_div, 2).reshape(1, D)

    out = pl.pallas_call(
        _body,
        grid=(grid,),
        in_specs=[
            pl.BlockSpec((BB, L), lambda i: (i, 0)),
            pl.BlockSpec((BB,), lambda i: (i,)),
            pl.BlockSpec((BB,), lambda i: (i,)),
            pl.BlockSpec((_V, D), lambda i: (0, 0)),
            pl.BlockSpec((_V, D), lambda i: (0, 0)),
            pl.BlockSpec((_CPAD, D), lambda i: (0, 0)),
            pl.BlockSpec((1, D), lambda i: (0, 0)),
            pl.BlockSpec((1, D), lambda i: (0, 0)),
            pl.BlockSpec((1, D), lambda i: (0, 0)),
            pl.BlockSpec((1, D), lambda i: (0, 0)),
            pl.BlockSpec((1, D), lambda i: (0, 0)),
        ],
        out_specs=pl.BlockSpec((L, BB, D), lambda i: (0, i, 0)),
        out_shape=jax.ShapeDtypeStruct((L, B, D), jnp.float32),
    )(y.astype(jnp.int32), charge.astype(jnp.int32), mz, pe32, emb_w, chw,
      ln1_g.reshape(1, D), ln1_b.reshape(1, D),
      ln2_g.reshape(1, D), ln2_b.reshape(1, D), mzd)
    return jnp.transpose(out, (1, 0, 2))
